# baseline (speedup 1.0000x reference)
import jax
import jax.numpy as jnp
from jax import lax
from jax.experimental import pallas as pl
from jax.experimental.pallas import tpu as pltpu
from jax.experimental.pallas import tpu_sc as plsc

B_ = 4096
C_ = 1000
L_ = 50
NC_ = 2
NS_ = 16
NW_ = NC_ * NS_
BPW_ = B_ // NW_
EPW_ = BPW_ * L_
SLAB_ = C_ * L_
NGRP_ = (L_ + 15) // 16


def _sc_onehot(tgt_hbm, out_hbm, buf0, buf1, tgt_v, sem0, sem1):
    wid = lax.axis_index("s") * NC_ + lax.axis_index("c")
    base_b = wid * BPW_
    base_e = wid * EPW_

    def zbody(r, _):
        buf0[0, pl.ds(r * 16, 16)] = jnp.zeros((16,), jnp.int32)
        buf1[0, pl.ds(r * 16, 16)] = jnp.zeros((16,), jnp.int32)
        return 0
    lax.fori_loop(0, SLAB_ // 16, zbody, 0)

    pltpu.sync_copy(tgt_hbm.at[pl.ds(base_e, EPW_)], tgt_v.at[pl.ds(0, EPW_)])

    lanes = lax.iota(jnp.int32, 16)
    row0 = jnp.zeros((16,), jnp.int32)

    def scatter_slab(buf, s, vals):
        for g in range(NGRP_):
            l = g * 16 + lanes
            mask = l < L_
            t = tgt_v[pl.ds(s * L_ + g * 16, 16)]
            off = t * L_ + l
            plsc.store_scatter(buf, [row0, off], vals, mask=mask)

    ones = jnp.ones((16,), jnp.int32)
    zeros = jnp.zeros((16,), jnp.int32)

    def dst(s):
        return out_hbm.at[pl.ds(base_b + s, 1), :]

    def body(i, _):
        for q, (buf, sem) in enumerate(((buf0, sem0), (buf1, sem1))):
            s = 2 * i + q

            @pl.when(i > 0)
            def _():
                pltpu.make_async_copy(buf, dst(s - 2), sem).wait()
                scatter_slab(buf, s - 2, zeros)
            scatter_slab(buf, s, ones)
            pltpu.make_async_copy(buf, dst(s), sem).start()
        return 0
    lax.fori_loop(0, BPW_ // 2, body, 0)

    pltpu.make_async_copy(buf0, dst(BPW_ - 2), sem0).wait()
    pltpu.make_async_copy(buf1, dst(BPW_ - 1), sem1).wait()


@jax.jit
def kernel(target):
    tgt_flat = jnp.reshape(target, (B_ * L_,))
    out2d = pl.kernel(
        _sc_onehot,
        out_type=jax.ShapeDtypeStruct((B_, SLAB_), jnp.int32),
        mesh=plsc.VectorSubcoreMesh(core_axis_name="c", subcore_axis_name="s"),
        compiler_params=pltpu.CompilerParams(needs_layout_passes=False),
        scratch_types=[
            pltpu.VMEM((1, SLAB_), jnp.int32),
            pltpu.VMEM((1, SLAB_), jnp.int32),
            pltpu.VMEM((EPW_ + 16,), jnp.int32),
            pltpu.SemaphoreType.DMA,
            pltpu.SemaphoreType.DMA,
        ],
    )(tgt_flat)
    return jnp.reshape(out2d, (B_, C_, L_))


# speedup vs baseline: 1.0905x; 1.0905x over previous
import jax
import jax.numpy as jnp
from jax import lax
from jax.experimental import pallas as pl
from jax.experimental.pallas import tpu as pltpu
from jax.experimental.pallas import tpu_sc as plsc

B_ = 4096
C_ = 1000
L_ = 50
NC_ = 2
NS_ = 16
NW_ = NC_ * NS_
BPW_ = B_ // NW_
EPW_ = BPW_ * L_
SLAB_ = C_ * L_
FROW_ = 128
ROWS_ = B_ * SLAB_ // FROW_
RPW_ = ROWS_ // NW_
CROWS_ = 400
CWORDS_ = CROWS_ * FROW_
NCH_ = RPW_ // CROWS_
NGRP_ = (L_ + 15) // 16
TPAD_ = EPW_ + 3 * L_ + 16


def _sc_onehot(tgt_hbm, out_hbm, buf0, buf1, tgt_v, sem0, sem1):
    wid = lax.axis_index("s") * NC_ + lax.axis_index("c")
    base_b = wid * BPW_
    base_e = wid * EPW_
    base_r = wid * RPW_

    def zbody(r, _):
        for g in range(FROW_ // 16):
            buf0[r, pl.ds(g * 16, 16)] = jnp.zeros((16,), jnp.int32)
            buf1[r, pl.ds(g * 16, 16)] = jnp.zeros((16,), jnp.int32)
        return 0
    lax.fori_loop(0, CROWS_, zbody, 0)

    pltpu.sync_copy(tgt_hbm.at[pl.ds(base_e, EPW_)], tgt_v.at[pl.ds(0, EPW_)])

    lanes = lax.iota(jnp.int32, 16)
    ones = jnp.ones((16,), jnp.int32)
    zeros = jnp.zeros((16,), jnp.int32)

    def scatter_chunk(buf, c, vals):
        s0 = (base_r + c * CROWS_) * FROW_
        b0 = lax.div(s0, SLAB_)
        for cand in range(3):
            bl = b0 + cand - base_b
            in_tile = bl < BPW_
            sbase = (b0 + cand) * SLAB_ - s0
            for g in range(NGRP_):
                l = g * 16 + lanes
                t = tgt_v[pl.ds(bl * L_ + g * 16, 16)]
                off = sbase + t * L_ + l
                mask = ((l < L_) & (off >= 0) & (off < CWORDS_)
                        & jnp.full((16,), in_tile))
                offc = jnp.maximum(off, 0)
                plsc.store_scatter(
                    buf, [lax.shift_right_logical(offc, 7), offc & (FROW_ - 1)],
                    vals, mask=mask)

    def fire(buf, sem, c):
        pltpu.make_async_copy(
            buf, out_hbm.at[pl.ds(base_r + c * CROWS_, CROWS_), :],
            sem).start()

    def wait(buf, sem, c):
        pltpu.make_async_copy(
            buf, out_hbm.at[pl.ds(base_r + c * CROWS_, CROWS_), :],
            sem).wait()

    def body(i, _):
        @pl.when(i > 0)
        def _():
            wait(buf0, sem0, 2 * i - 2)
            scatter_chunk(buf0, 2 * i - 2, zeros)
        scatter_chunk(buf0, 2 * i, ones)
        fire(buf0, sem0, 2 * i)

        @pl.when(2 * i + 1 < NCH_)
        def _():
            @pl.when(i > 0)
            def _():
                wait(buf1, sem1, 2 * i - 1)
                scatter_chunk(buf1, 2 * i - 1, zeros)
            scatter_chunk(buf1, 2 * i + 1, ones)
            fire(buf1, sem1, 2 * i + 1)
        return 0
    lax.fori_loop(0, (NCH_ + 1) // 2, body, 0)

    wait(buf0, sem0, NCH_ - 1)
    wait(buf1, sem1, NCH_ - 2)


@jax.jit
def kernel(target):
    tgt_flat = jnp.reshape(target, (B_ * L_,))
    out2d = pl.kernel(
        _sc_onehot,
        out_type=jax.ShapeDtypeStruct((ROWS_, FROW_), jnp.int32),
        mesh=plsc.VectorSubcoreMesh(core_axis_name="c", subcore_axis_name="s"),
        compiler_params=pltpu.CompilerParams(needs_layout_passes=False),
        scratch_types=[
            pltpu.VMEM((CROWS_, FROW_), jnp.int32),
            pltpu.VMEM((CROWS_, FROW_), jnp.int32),
            pltpu.VMEM((TPAD_,), jnp.int32),
            pltpu.SemaphoreType.DMA,
            pltpu.SemaphoreType.DMA,
        ],
    )(tgt_flat)
    return jnp.reshape(out2d, (B_, C_, L_))
